# baseline (speedup 1.0000x reference)
import functools

import jax
import jax.numpy as jnp
from jax import lax
from jax.experimental import pallas as pl
from jax.experimental.pallas import tpu as pltpu, tpu_sc as plsc

NC = 2
NS = 16
LANES = 16
NW = NC * NS


def _make_kernel(n_shots, n_seasons, max_weeks):
    b_per_w = n_shots // NW
    CH = 16384
    n_sub = b_per_w // CH
    row = n_seasons * max_weeks

    mesh = plsc.VectorSubcoreMesh(
        core_axis_name="c", subcore_axis_name="s",
        num_cores=NC, num_subcores=NS)

    @functools.partial(
        pl.kernel,
        out_type=jax.ShapeDtypeStruct((n_shots,), jnp.float32),
        mesh=mesh,
        scratch_types=[
            pltpu.VMEM((CH,), jnp.int32),
            pltpu.VMEM((CH,), jnp.int32),
            pltpu.VMEM((CH,), jnp.int32),
            pltpu.VMEM((CH,), jnp.int32),
            pltpu.VMEM((CH,), jnp.float32),
            pltpu.VMEM((CH,), jnp.float32),
            pltpu.VMEM((CH,), jnp.float32),
            pltpu.VMEM((2 * LANES,), jnp.float32),
            pltpu.SemaphoreType.DMA,
            pltpu.SemaphoreType.DMA,
        ],
    )
    def irt_kernel(theta_h, phi_h, xg_h, scal_h, sh_h, go_h, se_h, wk_h,
                   out_h,
                   sh_v, go_v, se_v, wk_v, th_v, ph_v, xg_v, scal_v,
                   sem_in, sem_g):
        wid = lax.axis_index("s") * NC + lax.axis_index("c")
        base = wid * b_per_w

        pltpu.sync_copy(scal_h, scal_v)
        b0v = scal_v[pl.ds(0, LANES)]
        alv = scal_v[pl.ds(LANES, LANES)]

        for c in range(n_sub):
            off = base + c * CH
            cps = [
                pltpu.async_copy(sh_h.at[pl.ds(off, CH)], sh_v, sem_in),
                pltpu.async_copy(go_h.at[pl.ds(off, CH)], go_v, sem_in),
                pltpu.async_copy(se_h.at[pl.ds(off, CH)], se_v, sem_in),
                pltpu.async_copy(wk_h.at[pl.ds(off, CH)], wk_v, sem_in),
                pltpu.async_copy(xg_h.at[pl.ds(off, CH)], xg_v, sem_in),
            ]
            for cp in cps:
                cp.wait()

            def ixbody(j, carry):
                st = pl.multiple_of(j * LANES, LANES)
                sl = pl.ds(st, LANES)
                offv = se_v[sl] * max_weeks + wk_v[sl]
                sh_v[sl] = sh_v[sl] * row + offv
                go_v[sl] = go_v[sl] * row + offv
                return carry
            lax.fori_loop(0, CH // LANES, ixbody, 0)

            g1 = pltpu.async_copy(theta_h.at[sh_v], th_v, sem_g)
            g2 = pltpu.async_copy(phi_h.at[go_v], ph_v, sem_g)
            g1.wait()
            g2.wait()

            def cmb(j, carry):
                st = pl.multiple_of(j * LANES, LANES)
                sl = pl.ds(st, LANES)
                xg_v[sl] = b0v + alv * xg_v[sl] + th_v[sl] - ph_v[sl]
                return carry
            lax.fori_loop(0, CH // LANES, cmb, 0)

            pltpu.sync_copy(xg_v, out_h.at[pl.ds(off, CH)])

    return irt_kernel


def kernel(xg_logit, theta, phi, beta0, alpha, shooter_idx, goalie_idx,
           season_idx, week_idx):
    n_shooters, n_seasons, max_weeks = theta.shape
    n_shots = xg_logit.shape[0]
    scal = jnp.concatenate([
        jnp.broadcast_to(beta0, (LANES,)),
        jnp.broadcast_to(alpha, (LANES,)),
    ])
    k = _make_kernel(n_shots, n_seasons, max_weeks)
    return k(theta.reshape(-1), phi.reshape(-1), xg_logit, scal,
             shooter_idx, goalie_idx, season_idx, week_idx)


# speedup vs baseline: 1.0210x; 1.0210x over previous
import functools

import jax
import jax.numpy as jnp
from jax import lax
from jax.experimental import pallas as pl
from jax.experimental.pallas import tpu as pltpu, tpu_sc as plsc

NC = 2
NS = 16
LANES = 16
NW = NC * NS


def _make_kernel(n_shots, n_seasons, max_weeks):
    b_per_w = n_shots // NW
    CH = 16384
    n_sub = b_per_w // CH
    row = n_seasons * max_weeks

    mesh = plsc.VectorSubcoreMesh(
        core_axis_name="c", subcore_axis_name="s",
        num_cores=NC, num_subcores=NS)

    @functools.partial(
        pl.kernel,
        out_type=jax.ShapeDtypeStruct((n_shots,), jnp.float32),
        mesh=mesh,
        scratch_types=[
            pltpu.VMEM((CH,), jnp.int32),
            pltpu.VMEM((CH,), jnp.int32),
            pltpu.VMEM((CH,), jnp.int32),
            pltpu.VMEM((CH,), jnp.int32),
            pltpu.VMEM((CH,), jnp.float32),
            pltpu.VMEM((CH,), jnp.float32),
            pltpu.VMEM((CH,), jnp.float32),
            pltpu.VMEM((2 * LANES,), jnp.float32),
            pltpu.SemaphoreType.DMA,
            pltpu.SemaphoreType.DMA,
        ],
    )
    def irt_kernel(theta_h, phi_h, xg_h, scal_h, sh_h, go_h, se_h, wk_h,
                   out_h,
                   sh_v, go_v, se_v, wk_v, th_v, ph_v, xg_v, scal_v,
                   sem_in, sem_g):
        wid = lax.axis_index("s") * NC + lax.axis_index("c")
        base = wid * b_per_w

        pltpu.sync_copy(scal_h, scal_v)
        b0v = scal_v[pl.ds(0, LANES)]
        alv = scal_v[pl.ds(LANES, LANES)]

        for c in range(n_sub):
            off = base + c * CH
            cps = [
                pltpu.async_copy(sh_h.at[pl.ds(off, CH)], sh_v, sem_in),
                pltpu.async_copy(go_h.at[pl.ds(off, CH)], go_v, sem_in),
                pltpu.async_copy(se_h.at[pl.ds(off, CH)], se_v, sem_in),
                pltpu.async_copy(wk_h.at[pl.ds(off, CH)], wk_v, sem_in),
                pltpu.async_copy(xg_h.at[pl.ds(off, CH)], xg_v, sem_in),
            ]
            for cp in cps:
                cp.wait()

            @plsc.parallel_loop(0, CH, LANES, unroll=8)
            def ixbody(i):
                sl = pl.ds(pl.multiple_of(i, LANES), LANES)
                offv = se_v[sl] * max_weeks + wk_v[sl]
                sh_v[sl] = sh_v[sl] * row + offv
                go_v[sl] = go_v[sl] * row + offv

            g1 = pltpu.async_copy(theta_h.at[sh_v], th_v, sem_g)
            g2 = pltpu.async_copy(phi_h.at[go_v], ph_v, sem_g)
            g1.wait()
            g2.wait()

            @plsc.parallel_loop(0, CH, LANES, unroll=8)
            def cmb(i):
                sl = pl.ds(pl.multiple_of(i, LANES), LANES)
                xg_v[sl] = b0v + alv * xg_v[sl] + th_v[sl] - ph_v[sl]

            pltpu.sync_copy(xg_v, out_h.at[pl.ds(off, CH)])

    return irt_kernel


def kernel(xg_logit, theta, phi, beta0, alpha, shooter_idx, goalie_idx,
           season_idx, week_idx):
    n_shooters, n_seasons, max_weeks = theta.shape
    n_shots = xg_logit.shape[0]
    scal = jnp.concatenate([
        jnp.broadcast_to(beta0, (LANES,)),
        jnp.broadcast_to(alpha, (LANES,)),
    ])
    k = _make_kernel(n_shots, n_seasons, max_weeks)
    return k(theta.reshape(-1), phi.reshape(-1), xg_logit, scal,
             shooter_idx, goalie_idx, season_idx, week_idx)
